# baseline (speedup 1.0000x reference)
import functools

import jax
import jax.numpy as jnp
from jax import lax
from jax.experimental import pallas as pl
from jax.experimental.pallas import tpu as pltpu
from jax.experimental.pallas import tpu_sc as plsc

BATCH = 16384
EMBED_DIM = 64
NUM_ROWS = 100000

NC = 2
NS = 16
NW = NC * NS
PIECE = 1280
NPIECES = 79
TAIL = NUM_ROWS - (NPIECES - 1) * PIECE
NP_MAX = 3
SLAB = 4096
NSLABS = BATCH // SLAB
GRAN = 16
VROWS = BATCH + NW

B_PER_W = BATCH // NW
BCHUNK = 256


def _a_body(uid_hbm, aid_hbm, ut_hbm, at_hbm, uvec_hbm, avec_hbm,
            ids_v, piece_v, tail_v, hloc_v, hitem_v, stag_v, sidx_v, sem):
    wid = lax.axis_index("s") * NC + lax.axis_index("c")
    lane = lax.iota(jnp.int32, 16)
    dump = BATCH + wid

    def do_piece(p, tab_hbm, ids_hbm, vec_hbm, buf_v, width):
        pltpu.sync_copy(ids_hbm, ids_v)
        off = pl.multiple_of(p * PIECE, 128)
        if width == PIECE:
            pltpu.sync_copy(tab_hbm.at[:, pl.ds(off, PIECE)], buf_v)
        else:
            pltpu.sync_copy(
                tab_hbm.at[:, pl.ds((NPIECES - 1) * PIECE, width)], buf_v)

        for s in range(NSLABS):
            def scan_step(k, n):
                idv = ids_v[pl.ds(s * SLAB + k * 16, 16)]
                loc = idv - off
                m = (loc >= 0) & (loc < PIECE)
                plsc.store_compressed(hloc_v.at[pl.ds(n, 16)], loc, mask=m)
                plsc.store_compressed(
                    hitem_v.at[pl.ds(n, 16)],
                    jnp.full((16,), s * SLAB + k * 16, jnp.int32) + lane,
                    mask=m)
                return n + plsc.all_reduce_population_count(m)[0]

            n = lax.fori_loop(0, SLAB // 16, scan_step, 0, unroll=False)

            def gran_step(b, carry):
                hsl = pl.ds(b * GRAN, 16)
                loc = jnp.minimum(hloc_v[hsl], width - 1)
                itm = hitem_v[hsl]
                valid = (b * GRAN + lane) < n
                sidx_v[0, :] = lax.select(
                    valid, itm, jnp.full((16,), dump, jnp.int32))
                d = lane
                for t in range(EMBED_DIM):
                    if t:
                        d = (d + 1) & (EMBED_DIM - 1)
                    plsc.store_scatter(
                        stag_v, [lane, d],
                        plsc.load_gather(buf_v, [d, loc]))
                pltpu.async_copy(stag_v, vec_hbm.at[sidx_v.at[0]], sem).wait()
                return carry

            lax.fori_loop(0, (n + GRAN - 1) // GRAN, gran_step, 0,
                          unroll=False)

    def piece_loop(i, carry):
        p = wid + NW * i

        @pl.when(p < NPIECES - 1)
        def _():
            do_piece(p, ut_hbm, uid_hbm, uvec_hbm, piece_v, PIECE)
            do_piece(p, at_hbm, aid_hbm, avec_hbm, piece_v, PIECE)

        @pl.when(p == NPIECES - 1)
        def _():
            do_piece(p, ut_hbm, uid_hbm, uvec_hbm, tail_v, TAIL)
            do_piece(p, at_hbm, aid_hbm, avec_hbm, tail_v, TAIL)

        return carry

    lax.fori_loop(0, NP_MAX, piece_loop, 0, unroll=False)


def _b_body(uvec_hbm, avec_hbm, out_hbm, ubuf_v, abuf_v, out_v, sem):
    wid = lax.axis_index("s") * NC + lax.axis_index("c")
    base = wid * B_PER_W
    lane = lax.iota(jnp.int32, 16)

    for h in range(B_PER_W // BCHUNK):
        pltpu.sync_copy(
            uvec_hbm.at[pl.ds(base + h * BCHUNK, BCHUNK), :], ubuf_v)
        pltpu.sync_copy(
            avec_hbm.at[pl.ds(base + h * BCHUNK, BCHUNK), :], abuf_v)

        def group_step(g, carry):
            row = g * 16 + lane
            d = lane
            acc = plsc.load_gather(ubuf_v, [row, d]) * plsc.load_gather(
                abuf_v, [row, d])
            for t in range(1, EMBED_DIM):
                d = (d + 1) & (EMBED_DIM - 1)
                acc = acc + plsc.load_gather(ubuf_v, [row, d]) * (
                    plsc.load_gather(abuf_v, [row, d]))
            out_v[pl.ds(h * BCHUNK + g * 16, 16)] = 1.0 / (1.0 + jnp.exp(-acc))
            return carry

        lax.fori_loop(0, BCHUNK // 16, group_step, 0, unroll=False)

    pltpu.sync_copy(out_v, out_hbm.at[pl.ds(base, B_PER_W)])


@jax.jit
def _run(user_ids, anime_ids, user_table, anime_table):
    mesh = plsc.VectorSubcoreMesh(core_axis_name="c", subcore_axis_name="s")
    cp = pltpu.CompilerParams(
        needs_layout_passes=False, use_tc_tiling_on_sc=True)
    ka = functools.partial(
        pl.kernel,
        mesh=mesh,
        compiler_params=cp,
        out_type=(jax.ShapeDtypeStruct((VROWS, 128), jnp.float32),
                  jax.ShapeDtypeStruct((VROWS, 128), jnp.float32)),
        scratch_types=[
            pltpu.VMEM((BATCH,), jnp.int32),
            pltpu.VMEM((EMBED_DIM, PIECE), jnp.float32),
            pltpu.VMEM((EMBED_DIM, TAIL), jnp.float32),
            pltpu.VMEM((SLAB,), jnp.int32),
            pltpu.VMEM((SLAB,), jnp.int32),
            pltpu.VMEM((GRAN, 128), jnp.float32),
            pltpu.VMEM((1, GRAN), jnp.int32),
            pltpu.SemaphoreType.DMA,
        ],
    )(_a_body)
    uvec, avec = ka(user_ids, anime_ids, user_table.T, anime_table.T)
    kb = functools.partial(
        pl.kernel,
        mesh=mesh,
        compiler_params=cp,
        out_type=jax.ShapeDtypeStruct((BATCH,), jnp.float32),
        scratch_types=[
            pltpu.VMEM((BCHUNK, 128), jnp.float32),
            pltpu.VMEM((BCHUNK, 128), jnp.float32),
            pltpu.VMEM((B_PER_W,), jnp.float32),
            pltpu.SemaphoreType.DMA,
        ],
    )(_b_body)
    return kb(uvec, avec)


def kernel(user_ids, anime_ids, user_table, anime_table):
    return _run(jnp.asarray(user_ids, jnp.int32), jnp.asarray(anime_ids, jnp.int32),
                user_table, anime_table)


# speedup vs baseline: 1.4934x; 1.4934x over previous
import functools

import jax
import jax.numpy as jnp
from jax import lax
from jax.experimental import pallas as pl
from jax.experimental.pallas import tpu as pltpu
from jax.experimental.pallas import tpu_sc as plsc

BATCH = 16384
EMBED_DIM = 64
NUM_ROWS = 100000
TBLK = 2048
TGRID = pl.cdiv(NUM_ROWS, TBLK)
PAD_ROWS = TGRID * TBLK

NC = 2
NS = 16
NW = NC * NS
B_PER_W = BATCH // NW
CHUNK = 128
N_CHUNKS = B_PER_W // CHUNK
GROUP = 16
GROUPS_PER_CHUNK = CHUNK // GROUP


def _pack_body(t_ref, out_ref):
    eye = (lax.broadcasted_iota(jnp.int32, (EMBED_DIM, EMBED_DIM), 0) ==
           lax.broadcasted_iota(jnp.int32, (EMBED_DIM, EMBED_DIM), 1)
           ).astype(jnp.float32)
    t = lax.dot_general(t_ref[...], eye, (((0,), (0,)), ((), ())),
                        preferred_element_type=jnp.float32)
    out_ref[:, 0:EMBED_DIM] = t
    out_ref[:, EMBED_DIM:2 * EMBED_DIM] = jnp.zeros(
        (TBLK, EMBED_DIM), jnp.float32)


def _pack(table_t):
    return pl.pallas_call(
        _pack_body,
        grid=(TGRID,),
        in_specs=[pl.BlockSpec((EMBED_DIM, TBLK), lambda j: (0, j))],
        out_specs=pl.BlockSpec((TBLK, 2 * EMBED_DIM), lambda j: (j, 0)),
        out_shape=jax.ShapeDtypeStruct((PAD_ROWS, 2 * EMBED_DIM), jnp.float32),
        compiler_params=pltpu.CompilerParams(fuse_transposed_lhs_in_matmul=True),
    )(table_t)


def _body(uid_hbm, aid_hbm, ut_hbm, at_hbm, out_hbm,
          uidx_v, aidx_v, ubuf_v, abuf_v, out_v, sem):
    wid = lax.axis_index("s") * NC + lax.axis_index("c")
    base = wid * B_PER_W

    pltpu.sync_copy(uid_hbm.at[pl.ds(base, B_PER_W)], uidx_v)
    pltpu.sync_copy(aid_hbm.at[pl.ds(base, B_PER_W)], aidx_v)

    lane = lax.iota(jnp.int32, 16)

    def chunk_step(c, carry):
        csl = pl.ds(c * CHUNK, CHUNK)
        cu = pltpu.async_copy(ut_hbm.at[uidx_v.at[csl]], ubuf_v, sem)
        ca = pltpu.async_copy(at_hbm.at[aidx_v.at[csl]], abuf_v, sem)
        cu.wait()
        ca.wait()

        def group_step(g, carry2):
            row = g * GROUP + lane
            d = lane
            acc = plsc.load_gather(ubuf_v, [row, d]) * plsc.load_gather(
                abuf_v, [row, d])
            for _ in range(1, EMBED_DIM):
                d = (d + 1) & (EMBED_DIM - 1)
                acc = acc + plsc.load_gather(ubuf_v, [row, d]) * (
                    plsc.load_gather(abuf_v, [row, d]))
            out_v[pl.ds(c * CHUNK + g * GROUP, 16)] = 1.0 / (1.0 + jnp.exp(-acc))
            return carry2

        lax.fori_loop(0, GROUPS_PER_CHUNK, group_step, 0)
        return carry

    lax.fori_loop(0, N_CHUNKS, chunk_step, 0)

    pltpu.sync_copy(out_v, out_hbm.at[pl.ds(base, B_PER_W)])


@jax.jit
def _run(user_ids, anime_ids, user_table, anime_table):
    ut2 = _pack(user_table.T)
    at2 = _pack(anime_table.T)
    mesh = plsc.VectorSubcoreMesh(core_axis_name="c", subcore_axis_name="s")
    k = functools.partial(
        pl.kernel,
        mesh=mesh,
        compiler_params=pltpu.CompilerParams(
            needs_layout_passes=False, use_tc_tiling_on_sc=True),
        out_type=jax.ShapeDtypeStruct((BATCH,), jnp.float32),
        scratch_types=[
            pltpu.VMEM((B_PER_W,), jnp.int32),
            pltpu.VMEM((B_PER_W,), jnp.int32),
            pltpu.VMEM((CHUNK, 2 * EMBED_DIM), jnp.float32),
            pltpu.VMEM((CHUNK, 2 * EMBED_DIM), jnp.float32),
            pltpu.VMEM((B_PER_W,), jnp.float32),
            pltpu.SemaphoreType.DMA,
        ],
    )(_body)
    return k(user_ids, anime_ids, ut2, at2)


def kernel(user_ids, anime_ids, user_table, anime_table):
    return _run(jnp.asarray(user_ids, jnp.int32), jnp.asarray(anime_ids, jnp.int32),
                user_table, anime_table)
